# baseline (speedup 1.0000x reference)
import functools

import jax
import jax.numpy as jnp
from jax.experimental import pallas as pl
from jax.experimental.pallas import tpu as pltpu

_N_NBR = 160000
_N_NODES = 10000
_F = 128
_H = 256
_E = 256

_RBLK = 2000
_NR = _N_NBR // _RBLK
_DBLK = 2000
_ND = _N_NODES // _DBLK


def _ln_relu(x, g, b, eps=1e-5):
    mu = jnp.mean(x, axis=-1, keepdims=True)
    var = jnp.mean((x - mu) ** 2, axis=-1, keepdims=True)
    y = (x - mu) / jnp.sqrt(var + eps) * g + b
    return jnp.maximum(y, 0.0)


def _body(l1_ref, l2_ref, nf_ref, ws1_ref, wn1_ref, c1b_ref, g1_ref, be1_ref,
          ws2_ref, wn2_ref, c2b_ref, g2_ref, be2_ref, out_ref,
          s1, s2, ssq, h2s):
    i = pl.program_id(0)

    @pl.when(i == 0)
    def _():
        s1[...] = jnp.zeros_like(s1)
        s2[...] = jnp.zeros_like(s2)
        ssq[...] = jnp.zeros_like(ssq)

    @pl.when(i < _NR)
    def _():
        s1[...] += jnp.sum(l1_ref[...], axis=0, keepdims=True)
        s2[...] += jnp.sum(l2_ref[...], axis=0, keepdims=True)

    @pl.when((i >= _NR) & (i < _NR + _ND))
    def _():
        j = i - _NR
        inv = 1.0 / _N_NBR
        agg1 = s1[...] * inv
        agg2 = s2[...] * inv
        c1 = jnp.dot(agg1, wn1_ref[...], preferred_element_type=jnp.float32) + c1b_ref[...]
        out1 = jnp.dot(nf_ref[...], ws1_ref[...], preferred_element_type=jnp.float32) + c1
        h1 = _ln_relu(out1, g1_ref[...], be1_ref[...])
        c2 = jnp.dot(agg2, wn2_ref[...], preferred_element_type=jnp.float32) + c2b_ref[...]
        out2 = jnp.dot(h1, ws2_ref[...], preferred_element_type=jnp.float32) + c2
        h2 = _ln_relu(out2, g2_ref[...], be2_ref[...])
        h2s[pl.ds(j * _DBLK, _DBLK), :] = h2
        ssq[...] += jnp.sum(h2 * h2, axis=0, keepdims=True)

    @pl.when(i >= _NR + _ND)
    def _():
        j = i - _NR - _ND
        norm = jnp.sqrt(ssq[...])
        out_ref[...] = h2s[pl.ds(j * _DBLK, _DBLK), :] / jnp.maximum(norm, 1e-12)


@jax.jit
def kernel(node_feat, neighbor_feats_l1, neighbor_feats_l2, W_self1, b_self1,
           W_nbr1, b_nbr1, g1, be1, W_self2, b_self2, W_nbr2, b_nbr2, g2, be2):
    f32 = jnp.float32
    c1b = (b_self1 + b_nbr1).reshape(1, _H)
    c2b = (b_self2 + b_nbr2).reshape(1, _E)

    grid = (_NR + 2 * _ND,)

    def _clamp(lo, x, hi):
        return jnp.minimum(jnp.maximum(x, lo), hi)

    out = pl.pallas_call(
        _body,
        grid=grid,
        in_specs=[
            pl.BlockSpec((_RBLK, _F), lambda i: (_clamp(0, i, _NR - 1), 0)),
            pl.BlockSpec((_RBLK, _H), lambda i: (_clamp(0, i, _NR - 1), 0)),
            pl.BlockSpec((_DBLK, _F), lambda i: (_clamp(0, i - _NR, _ND - 1), 0)),
            pl.BlockSpec((_F, _H), lambda i: (0, 0)),
            pl.BlockSpec((_F, _H), lambda i: (0, 0)),
            pl.BlockSpec((1, _H), lambda i: (0, 0)),
            pl.BlockSpec((1, _H), lambda i: (0, 0)),
            pl.BlockSpec((1, _H), lambda i: (0, 0)),
            pl.BlockSpec((_H, _E), lambda i: (0, 0)),
            pl.BlockSpec((_H, _E), lambda i: (0, 0)),
            pl.BlockSpec((1, _E), lambda i: (0, 0)),
            pl.BlockSpec((1, _E), lambda i: (0, 0)),
            pl.BlockSpec((1, _E), lambda i: (0, 0)),
        ],
        out_specs=pl.BlockSpec((_DBLK, _E), lambda i: (_clamp(0, i - _NR - _ND, _ND - 1), 0)),
        out_shape=jax.ShapeDtypeStruct((_N_NODES, _E), f32),
        scratch_shapes=[
            pltpu.VMEM((1, _F), f32),
            pltpu.VMEM((1, _H), f32),
            pltpu.VMEM((1, _E), f32),
            pltpu.VMEM((_N_NODES, _E), f32),
        ],
    )(neighbor_feats_l1, neighbor_feats_l2, node_feat, W_self1, W_nbr1, c1b,
      g1.reshape(1, _H), be1.reshape(1, _H), W_self2, W_nbr2, c2b,
      g2.reshape(1, _E), be2.reshape(1, _E))

    return out
